# Initial kernel scaffold; baseline (speedup 1.0000x reference)
import jax
import jax.numpy as jnp
from jax.experimental import pallas as pl


def kernel(x, edge_index, W, b):
    raise NotImplementedError("write your pallas kernel here")


# speedup vs baseline: 2.8967x; 2.8967x over previous
import functools

import jax
import jax.numpy as jnp
from jax import lax
from jax.experimental import pallas as pl
from jax.experimental.pallas import tpu as pltpu
from jax.experimental.pallas import tpu_sc as plsc

N = 10000
E = 160000
D = 256
U = 256
NC = 2
NS = 16
L = 16
NP = 10240
EPAD = 163840
CH = 128
HALF = D // 2

_sc_mesh = plsc.VectorSubcoreMesh(core_axis_name="c", subcore_axis_name="s")


@functools.partial(
    pl.kernel,
    out_type=jax.ShapeDtypeStruct((NP,), jnp.float32),
    mesh=_sc_mesh,
    scratch_types=[
        pltpu.VMEM_SHARED((NP,), jnp.float32),
        pltpu.VMEM((CH,), jnp.int32),
        pltpu.VMEM((CH,), jnp.float32),
        pltpu.VMEM((NP // NS,), jnp.float32),
    ],
)
def _degree_kernel(dst_hbm, deg_hbm, deg_sh, dstbuf, ones, zbuf):
    c = lax.axis_index("c")
    s = lax.axis_index("s")
    tid = c * NS + s
    ept = EPAD // (NC * NS)
    zslice = NP // NS

    def fill(i, _):
        zbuf[pl.ds(i * L, L)] = jnp.zeros((L,), jnp.float32)
        return 0

    lax.fori_loop(0, zslice // L, fill, 0)
    for j in range(CH // L):
        ones[pl.ds(j * L, L)] = jnp.ones((L,), jnp.float32)

    pltpu.sync_copy(zbuf, deg_sh.at[pl.ds(s * zslice, zslice)])
    plsc.subcore_barrier()

    def body(g, _):
        b = tid * ept + g * CH
        pltpu.sync_copy(dst_hbm.at[pl.ds(b, CH)], dstbuf)
        pltpu.sync_copy(ones, deg_sh.at[dstbuf], add=True)
        return 0

    lax.fori_loop(0, ept // CH, body, 0)
    plsc.subcore_barrier()

    @pl.when(c == 0)
    def _():
        pltpu.sync_copy(deg_sh.at[pl.ds(s * zslice, zslice)],
                        deg_hbm.at[pl.ds(s * zslice, zslice)])


def _norm_body(x_ref, deg_ref, xs_ref, isd_ref):
    isd = lax.rsqrt(deg_ref[...])
    xn = isd * x_ref[...]
    xs_ref[0] = xn[:, :HALF]
    xs_ref[1] = xn[:, HALF:]
    isd_ref[...] = isd


def _norm_call(x, degc):
    R = 2000
    grid = (N // R,)
    return pl.pallas_call(
        _norm_body,
        grid=grid,
        in_specs=[
            pl.BlockSpec((R, D), lambda i: (i, 0)),
            pl.BlockSpec((R, 1), lambda i: (i, 0)),
        ],
        out_specs=[
            pl.BlockSpec((2, R, HALF), lambda i: (0, i, 0)),
            pl.BlockSpec((R, 1), lambda i: (i, 0)),
        ],
        out_shape=[
            jax.ShapeDtypeStruct((2, N, HALF), jnp.float32),
            jax.ShapeDtypeStruct((N, 1), jnp.float32),
        ],
    )(x, degc)


@functools.partial(
    pl.kernel,
    out_type=jax.ShapeDtypeStruct((NC * NP, HALF), jnp.float32),
    mesh=_sc_mesh,
    scratch_types=[
        pltpu.VMEM_SHARED((NP, HALF), jnp.float32),
        pltpu.VMEM((CH,), jnp.int32),
        pltpu.VMEM((CH,), jnp.int32),
        pltpu.VMEM((CH, HALF), jnp.float32),
        pltpu.VMEM((8, HALF), jnp.float32),
    ],
)
def _pool_kernel(src_hbm, dst_hbm, xs_hbm, out_hbm,
                 pooled_sh, srcbuf, dstbuf, rows, zrows):
    c = lax.axis_index("c")
    s = lax.axis_index("s")
    ept = EPAD // NS
    zslice = NP // NS
    off = c * N

    for i in range(8):
        for j in range(HALF // L):
            zrows[i, pl.ds(j * L, L)] = jnp.zeros((L,), jnp.float32)

    def zero(k, _):
        pltpu.sync_copy(zrows, pooled_sh.at[pl.ds(s * zslice + k * 8, 8), :])
        return 0

    lax.fori_loop(0, zslice // 8, zero, 0)
    plsc.subcore_barrier()

    def body(g, _):
        b = s * ept + g * CH
        pltpu.sync_copy(src_hbm.at[pl.ds(b, CH)], srcbuf)
        pltpu.sync_copy(dst_hbm.at[pl.ds(b, CH)], dstbuf)
        for j in range(CH // L):
            v = srcbuf[pl.ds(j * L, L)]
            srcbuf[pl.ds(j * L, L)] = v + off
        pltpu.sync_copy(xs_hbm.at[srcbuf], rows)
        pltpu.sync_copy(rows, pooled_sh.at[dstbuf], add=True)
        return 0

    lax.fori_loop(0, ept // CH, body, 0)
    plsc.subcore_barrier()

    pltpu.sync_copy(
        pooled_sh.at[pl.ds(s * zslice, zslice), :],
        out_hbm.at[pl.ds(c * NP + s * zslice, zslice), :],
    )


def _dense_body(p_ref, isd_ref, w_ref, b_ref, o_ref):
    acc = jnp.dot(p_ref[0], w_ref[0], preferred_element_type=jnp.float32)
    acc += jnp.dot(p_ref[1], w_ref[1], preferred_element_type=jnp.float32)
    o_ref[...] = jnp.maximum(isd_ref[...] * acc + b_ref[...], 0.0)


def _dense_call(pooled3, isd, w3, b2):
    R = 2000
    grid = (N // R,)
    return pl.pallas_call(
        _dense_body,
        grid=grid,
        in_specs=[
            pl.BlockSpec((2, R, HALF), lambda i: (0, i, 0)),
            pl.BlockSpec((R, 1), lambda i: (i, 0)),
            pl.BlockSpec((2, HALF, U), lambda i: (0, 0, 0)),
            pl.BlockSpec((1, U), lambda i: (0, 0)),
        ],
        out_specs=pl.BlockSpec((R, U), lambda i: (i, 0)),
        out_shape=jax.ShapeDtypeStruct((N, U), jnp.float32),
    )(pooled3, isd, w3, b2)


@jax.jit
def kernel(x, edge_index, W, b):
    src = edge_index[0]
    dst = edge_index[1]
    pad = EPAD - E
    src_pad = jnp.concatenate([src, jnp.zeros((pad,), jnp.int32)])
    dst_pad = jnp.concatenate([dst, jnp.full((pad,), NP - 1, jnp.int32)])

    deg = _degree_kernel(dst_pad)
    degc = deg[:N].reshape(N, 1)
    xs3, isd = _norm_call(x, degc)
    pooled = _pool_kernel(src_pad, dst_pad, xs3.reshape(2 * N, HALF))
    pooled3 = pooled.reshape(NC, NP, HALF)
    out = _dense_call(pooled3, isd, W.reshape(2, HALF, U), b.reshape(1, U))
    return out


# speedup vs baseline: 3.7674x; 1.3006x over previous
import functools

import jax
import jax.numpy as jnp
from jax import lax
from jax.experimental import pallas as pl
from jax.experimental.pallas import tpu as pltpu
from jax.experimental.pallas import tpu_sc as plsc

N = 10000
E = 160000
D = 256
U = 256
NC = 2
NS = 16
L = 16
NP = 10240
EPAD = 163840
CH = 128
HALF = D // 2

_sc_mesh = plsc.VectorSubcoreMesh(core_axis_name="c", subcore_axis_name="s")


@functools.partial(
    pl.kernel,
    out_type=jax.ShapeDtypeStruct((NP,), jnp.float32),
    mesh=_sc_mesh,
    scratch_types=[
        pltpu.VMEM_SHARED((NP,), jnp.float32),
        pltpu.VMEM((CH,), jnp.int32),
        pltpu.VMEM((CH,), jnp.float32),
        pltpu.VMEM((NP // NS,), jnp.float32),
    ],
)
def _degree_kernel(dst_hbm, deg_hbm, deg_sh, dstbuf, ones, zbuf):
    c = lax.axis_index("c")
    s = lax.axis_index("s")
    tid = c * NS + s
    ept = EPAD // (NC * NS)
    zslice = NP // NS

    def fill(i, _):
        zbuf[pl.ds(i * L, L)] = jnp.zeros((L,), jnp.float32)
        return 0

    lax.fori_loop(0, zslice // L, fill, 0)
    for j in range(CH // L):
        ones[pl.ds(j * L, L)] = jnp.ones((L,), jnp.float32)

    pltpu.sync_copy(zbuf, deg_sh.at[pl.ds(s * zslice, zslice)])
    plsc.subcore_barrier()

    def body(g, _):
        b = tid * ept + g * CH
        pltpu.sync_copy(dst_hbm.at[pl.ds(b, CH)], dstbuf)
        pltpu.sync_copy(ones, deg_sh.at[dstbuf], add=True)
        return 0

    lax.fori_loop(0, ept // CH, body, 0)
    plsc.subcore_barrier()

    @pl.when(c == 0)
    def _():
        pltpu.sync_copy(deg_sh.at[pl.ds(s * zslice, zslice)],
                        deg_hbm.at[pl.ds(s * zslice, zslice)])


def _norm_body(x_ref, deg_ref, xs_ref, isd_ref):
    isd = lax.rsqrt(deg_ref[...])
    xn = isd * x_ref[...]
    xs_ref[0] = xn[:, :HALF]
    xs_ref[1] = xn[:, HALF:]
    isd_ref[...] = isd


def _norm_call(x, degc):
    R = 2000
    grid = (N // R,)
    return pl.pallas_call(
        _norm_body,
        grid=grid,
        in_specs=[
            pl.BlockSpec((R, D), lambda i: (i, 0)),
            pl.BlockSpec((R, 1), lambda i: (i, 0)),
        ],
        out_specs=[
            pl.BlockSpec((2, R, HALF), lambda i: (0, i, 0)),
            pl.BlockSpec((R, 1), lambda i: (i, 0)),
        ],
        out_shape=[
            jax.ShapeDtypeStruct((2, N, HALF), jnp.float32),
            jax.ShapeDtypeStruct((N, 1), jnp.float32),
        ],
    )(x, degc)


NCHT = EPAD // NS // CH
NB = 2


@functools.partial(
    pl.kernel,
    out_type=jax.ShapeDtypeStruct((NC * NP, HALF), jnp.float32),
    mesh=_sc_mesh,
    scratch_types=[
        pltpu.VMEM_SHARED((NP, HALF), jnp.float32),
        pltpu.VMEM((NCHT, CH), jnp.int32),
        [pltpu.VMEM((1, CH), jnp.int32) for _ in range(NB)],
        [pltpu.VMEM((CH, HALF), jnp.float32) for _ in range(NB)],
        [pltpu.SemaphoreType.DMA for _ in range(NB)],
        [pltpu.SemaphoreType.DMA for _ in range(NB)],
        [pltpu.SemaphoreType.DMA for _ in range(NB)],
        pltpu.VMEM((8, HALF), jnp.float32),
    ],
)
def _pool_kernel(src_hbm, dst_hbm, xs_hbm, out_hbm,
                 pooled_sh, srcall, dstb, rows, gsem, dsem, ssem, zrows):
    c = lax.axis_index("c")
    s = lax.axis_index("s")
    zslice = NP // NS
    off = c * N

    for i in range(8):
        for j in range(HALF // L):
            zrows[i, pl.ds(j * L, L)] = jnp.zeros((L,), jnp.float32)

    def zero(k, _):
        pltpu.sync_copy(zrows, pooled_sh.at[pl.ds(s * zslice + k * 8, 8), :])
        return 0

    lax.fori_loop(0, zslice // 8, zero, 0)

    pltpu.sync_copy(src_hbm.at[pl.ds(s * NCHT, NCHT), :], srcall)

    def addoff(t, _):
        i = t // (CH // L)
        j = t % (CH // L)
        v = srcall[i, pl.ds(j * L, L)]
        srcall[i, pl.ds(j * L, L)] = v + off
        return 0

    lax.fori_loop(0, NCHT * (CH // L), addoff, 0)
    plsc.subcore_barrier()

    def start_gather(g, b):
        pltpu.async_copy(xs_hbm.at[srcall.at[g]], rows[b], gsem[b])

    def wait_gather(g, b):
        pltpu.make_async_copy(xs_hbm.at[srcall.at[g]], rows[b],
                              gsem[b]).wait()

    def start_dst(g, b):
        pltpu.async_copy(dst_hbm.at[pl.ds(s * NCHT + g, 1), :], dstb[b],
                         dsem[b])

    def wait_dst(g, b):
        pltpu.make_async_copy(dst_hbm.at[pl.ds(s * NCHT + g, 1), :],
                              dstb[b], dsem[b]).wait()

    def start_scatter(g, b):
        pltpu.async_copy(rows[b], pooled_sh.at[dstb[b].at[0]], ssem[b],
                         add=True)

    def wait_scatter(g, b):
        pltpu.make_async_copy(rows[b], pooled_sh.at[dstb[b].at[0]],
                              ssem[b]).wait()

    for bb in range(NB):
        start_dst(bb, bb)
        start_gather(bb, bb)

    def body(k, _):
        for bb in range(NB):
            g = k * NB + bb
            wait_gather(g, bb)
            wait_dst(g, bb)
            start_scatter(g, bb)
            gn = g + NB

            @pl.when(gn < NCHT)
            def _():
                wait_scatter(g, bb)
                start_dst(gn, bb)
                start_gather(gn, bb)
        return 0

    lax.fori_loop(0, NCHT // NB, body, 0)
    for bb in range(NB):
        wait_scatter(NCHT - NB + bb, bb)
    plsc.subcore_barrier()

    pltpu.sync_copy(
        pooled_sh.at[pl.ds(s * zslice, zslice), :],
        out_hbm.at[pl.ds(c * NP + s * zslice, zslice), :],
    )


def _dense_body(p_ref, isd_ref, w_ref, b_ref, o_ref):
    acc = jnp.dot(p_ref[0], w_ref[0], preferred_element_type=jnp.float32)
    acc += jnp.dot(p_ref[1], w_ref[1], preferred_element_type=jnp.float32)
    o_ref[...] = jnp.maximum(isd_ref[...] * acc + b_ref[...], 0.0)


def _dense_call(pooled3, isd, w3, b2):
    R = 2000
    grid = (N // R,)
    return pl.pallas_call(
        _dense_body,
        grid=grid,
        in_specs=[
            pl.BlockSpec((2, R, HALF), lambda i: (0, i, 0)),
            pl.BlockSpec((R, 1), lambda i: (i, 0)),
            pl.BlockSpec((2, HALF, U), lambda i: (0, 0, 0)),
            pl.BlockSpec((1, U), lambda i: (0, 0)),
        ],
        out_specs=pl.BlockSpec((R, U), lambda i: (i, 0)),
        out_shape=jax.ShapeDtypeStruct((N, U), jnp.float32),
    )(pooled3, isd, w3, b2)


@jax.jit
def kernel(x, edge_index, W, b):
    src = edge_index[0]
    dst = edge_index[1]
    pad = EPAD - E
    src_pad = jnp.concatenate([src, jnp.zeros((pad,), jnp.int32)])
    dst_pad = jnp.concatenate([dst, jnp.full((pad,), NP - 1, jnp.int32)])

    deg = _degree_kernel(dst_pad)
    degc = deg[:N].reshape(N, 1)
    xs3, isd = _norm_call(x, degc)
    pooled = _pool_kernel(src_pad.reshape(EPAD // CH, CH),
                          dst_pad.reshape(EPAD // CH, CH),
                          xs3.reshape(2 * N, HALF))
    pooled3 = pooled.reshape(NC, NP, HALF)
    out = _dense_call(pooled3, isd, W.reshape(2, HALF, U), b.reshape(1, U))
    return out
